# baseline (speedup 1.0000x reference)
import functools

import jax
import jax.numpy as jnp
from jax import lax
from jax.experimental import pallas as pl
from jax.experimental.pallas import tpu as pltpu
from jax.experimental.pallas import tpu_sc as plsc

_BATCH = 16384
_FEATS = 64


def _sc_gather_diff(home_idx, away_idx, strengths):
    info = plsc.get_sparse_core_info()
    num_workers = info.num_cores * info.num_subcores
    bpw = _BATCH // num_workers
    mesh = plsc.VectorSubcoreMesh(core_axis_name="c", subcore_axis_name="s")

    @functools.partial(
        pl.kernel,
        mesh=mesh,
        out_type=jax.ShapeDtypeStruct((_BATCH,), jnp.float32),
        scratch_types=[
            pltpu.VMEM((bpw,), jnp.int32),
            pltpu.VMEM((bpw,), jnp.int32),
            pltpu.VMEM((bpw,), jnp.float32),
            pltpu.VMEM((bpw,), jnp.float32),
            pltpu.VMEM((bpw,), jnp.float32),
            pltpu.SemaphoreType.DMA,
            pltpu.SemaphoreType.DMA,
        ],
    )
    def k(home_hbm, away_hbm, table_hbm, out_hbm, ih, ia, sh, sa, dv, s1, s2):
        wid = lax.axis_index("s") * info.num_cores + lax.axis_index("c")
        base = wid * bpw
        pltpu.sync_copy(home_hbm.at[pl.ds(base, bpw)], ih)
        pltpu.sync_copy(away_hbm.at[pl.ds(base, bpw)], ia)
        c1 = pltpu.async_copy(table_hbm.at[ih], sh, s1)
        c2 = pltpu.async_copy(table_hbm.at[ia], sa, s2)
        c1.wait()
        c2.wait()
        for i in range(bpw // 16):
            sl = pl.ds(i * 16, 16)
            dv[sl] = sh[sl] - sa[sl]
        pltpu.sync_copy(dv, out_hbm.at[pl.ds(base, bpw)])

    return k(home_idx, away_idx, strengths)


def _tc_body(x_ref, b_ref, m_ref, d_ref, o_ref):
    y = jnp.dot(x_ref[...], b_ref[...], preferred_element_type=jnp.float32)
    o_ref[...] = y + d_ref[...] + m_ref[0]


def _tc_combine(X, beta, mu, d):
    bm = 2048
    out = pl.pallas_call(
        _tc_body,
        grid=(_BATCH // bm,),
        in_specs=[
            pl.BlockSpec((bm, _FEATS), lambda i: (i, 0)),
            pl.BlockSpec((_FEATS, 1), lambda i: (0, 0)),
            pl.BlockSpec(memory_space=pltpu.SMEM),
            pl.BlockSpec((bm, 1), lambda i: (i, 0)),
        ],
        out_specs=pl.BlockSpec((bm, 1), lambda i: (i, 0)),
        out_shape=jax.ShapeDtypeStruct((_BATCH, 1), jnp.float32),
    )(X, beta.reshape(_FEATS, 1), mu, d.reshape(_BATCH, 1))
    return out.reshape(_BATCH)


@jax.jit
def kernel(home_idx, away_idx, X, strengths, beta, mu):
    d = _sc_gather_diff(home_idx, away_idx, strengths)
    return _tc_combine(X, beta, mu, d)


# speedup vs baseline: 1.3475x; 1.3475x over previous
import functools

import jax
import jax.numpy as jnp
from jax import lax
from jax.experimental import pallas as pl
from jax.experimental.pallas import tpu as pltpu
from jax.experimental.pallas import tpu_sc as plsc

_BATCH = 16384
_FEATS = 64


def _sc_gather_diff(home_idx, away_idx, strengths):
    info = plsc.get_sparse_core_info()
    num_workers = info.num_cores * info.num_subcores
    bpw = _BATCH // num_workers
    mesh = plsc.VectorSubcoreMesh(core_axis_name="c", subcore_axis_name="s")

    @functools.partial(
        pl.kernel,
        mesh=mesh,
        out_type=jax.ShapeDtypeStruct((_BATCH,), jnp.float32),
        scratch_types=[
            pltpu.VMEM((bpw,), jnp.int32),
            pltpu.VMEM((bpw,), jnp.int32),
            pltpu.VMEM((bpw,), jnp.float32),
            pltpu.VMEM((bpw,), jnp.float32),
            pltpu.VMEM((bpw,), jnp.float32),
            pltpu.SemaphoreType.DMA,
            pltpu.SemaphoreType.DMA,
        ],
    )
    def k(home_hbm, away_hbm, table_hbm, out_hbm, ih, ia, sh, sa, dv, s1, s2):
        wid = lax.axis_index("s") * info.num_cores + lax.axis_index("c")
        base = wid * bpw
        pltpu.sync_copy(home_hbm.at[pl.ds(base, bpw)], ih)
        pltpu.sync_copy(away_hbm.at[pl.ds(base, bpw)], ia)
        c1 = pltpu.async_copy(table_hbm.at[ih], sh, s1)
        c2 = pltpu.async_copy(table_hbm.at[ia], sa, s2)
        c1.wait()
        c2.wait()
        for i in range(bpw // 16):
            sl = pl.ds(i * 16, 16)
            dv[sl] = sh[sl] - sa[sl]
        pltpu.sync_copy(dv, out_hbm.at[pl.ds(base, bpw)])

    return k(home_idx, away_idx, strengths)


def _matvec_body(x_ref, b_ref, m_ref, o_ref):
    s = jnp.sum(x_ref[...] * b_ref[...], axis=1)
    o_ref[...] = s + m_ref[0]


def _tc_matvec(X, beta, mu):
    bm = 2048
    return pl.pallas_call(
        _matvec_body,
        grid=(_BATCH // bm,),
        in_specs=[
            pl.BlockSpec((bm, _FEATS), lambda i: (i, 0)),
            pl.BlockSpec((_FEATS,), lambda i: (0,)),
            pl.BlockSpec(memory_space=pltpu.SMEM),
        ],
        out_specs=pl.BlockSpec((bm,), lambda i: (i,)),
        out_shape=jax.ShapeDtypeStruct((_BATCH,), jnp.float32),
    )(X, beta, mu)


def _combine_body(y_ref, d_ref, o_ref):
    o_ref[...] = y_ref[...] + d_ref[...]


def _tc_combine(y, d):
    return pl.pallas_call(
        _combine_body,
        out_shape=jax.ShapeDtypeStruct((_BATCH,), jnp.float32),
    )(y, d)


@jax.jit
def kernel(home_idx, away_idx, X, strengths, beta, mu):
    d = _sc_gather_diff(home_idx, away_idx, strengths)
    y = _tc_matvec(X, beta, mu)
    return _tc_combine(y, d)


# speedup vs baseline: 1.7076x; 1.2673x over previous
import functools

import jax
import jax.numpy as jnp
from jax import lax
from jax.experimental import pallas as pl
from jax.experimental.pallas import tpu as pltpu
from jax.experimental.pallas import tpu_sc as plsc

_BATCH = 16384
_FEATS = 64


def _sc_gather_diff(home_idx, away_idx, strengths):
    info = plsc.get_sparse_core_info()
    num_workers = info.num_cores * info.num_subcores
    bpw = _BATCH // num_workers
    mesh = plsc.VectorSubcoreMesh(core_axis_name="c", subcore_axis_name="s")

    @functools.partial(
        pl.kernel,
        mesh=mesh,
        out_type=jax.ShapeDtypeStruct((_BATCH,), jnp.float32),
        scratch_types=[
            pltpu.VMEM((bpw,), jnp.int32),
            pltpu.VMEM((bpw,), jnp.int32),
            pltpu.VMEM((bpw,), jnp.float32),
            pltpu.VMEM((bpw,), jnp.float32),
            pltpu.VMEM((bpw,), jnp.float32),
            pltpu.SemaphoreType.DMA,
            pltpu.SemaphoreType.DMA,
        ],
    )
    def k(home_hbm, away_hbm, table_hbm, out_hbm, ih, ia, sh, sa, dv, s1, s2):
        wid = lax.axis_index("s") * info.num_cores + lax.axis_index("c")
        base = wid * bpw
        pltpu.sync_copy(home_hbm.at[pl.ds(base, bpw)], ih)
        pltpu.sync_copy(away_hbm.at[pl.ds(base, bpw)], ia)
        c1 = pltpu.async_copy(table_hbm.at[ih], sh, s1)
        c2 = pltpu.async_copy(table_hbm.at[ia], sa, s2)
        c1.wait()
        c2.wait()
        for i in range(bpw // 16):
            sl = pl.ds(i * 16, 16)
            dv[sl] = sh[sl] - sa[sl]
        pltpu.sync_copy(dv, out_hbm.at[pl.ds(base, bpw)])

    return k(home_idx, away_idx, strengths)


def _matvec_body(xt_ref, b_ref, m_ref, d_ref, o_ref):
    p = xt_ref[...] * b_ref[...][:, None]
    s = jnp.sum(p, axis=0)
    o_ref[...] = s + d_ref[...] + m_ref[0]


def _tc_matvec_combine(XT, beta, mu, d):
    bm = 2048
    return pl.pallas_call(
        _matvec_body,
        grid=(_BATCH // bm,),
        in_specs=[
            pl.BlockSpec((_FEATS, bm), lambda i: (0, i)),
            pl.BlockSpec((_FEATS,), lambda i: (0,)),
            pl.BlockSpec(memory_space=pltpu.SMEM),
            pl.BlockSpec((bm,), lambda i: (i,)),
        ],
        out_specs=pl.BlockSpec((bm,), lambda i: (i,)),
        out_shape=jax.ShapeDtypeStruct((_BATCH,), jnp.float32),
    )(XT, beta, mu, d)


@jax.jit
def kernel(home_idx, away_idx, X, strengths, beta, mu):
    d = _sc_gather_diff(home_idx, away_idx, strengths)
    XT = X.T
    return _tc_matvec_combine(XT, beta, mu, d)
